# baseline (speedup 1.0000x reference)
import functools

import jax
import jax.numpy as jnp
from jax import lax
from jax.experimental import pallas as pl
from jax.experimental.pallas import tpu as pltpu
from jax.experimental.pallas import tpu_sc as plsc

_NUM_CORES = 2
_NUM_SUBCORES = 16
_NUM_WORKERS = _NUM_CORES * _NUM_SUBCORES
_CHUNK = 1024


@functools.partial(jax.jit, static_argnames=("n_rows", "embed_dim"))
def _sc_gather(idx_flat, table, n_rows, embed_dim):
    rows_per_w = n_rows // _NUM_WORKERS
    n_chunks = rows_per_w // _CHUNK
    mesh = plsc.VectorSubcoreMesh(core_axis_name="c", subcore_axis_name="s")

    @functools.partial(
        pl.kernel,
        out_type=jax.ShapeDtypeStruct((n_rows, embed_dim), jnp.float32),
        mesh=mesh,
        scratch_types=[
            pltpu.VMEM((_CHUNK,), jnp.int32),
            pltpu.VMEM((_CHUNK, embed_dim), jnp.float32),
            pltpu.SemaphoreType.DMA,
        ],
        compiler_params=pltpu.CompilerParams(use_tc_tiling_on_sc=False),
    )
    def gather_kernel(idx_hbm, table_hbm, out_hbm, idx_v, rows_v, sem):
        wid = lax.axis_index("s") * _NUM_CORES + lax.axis_index("c")
        base = wid * rows_per_w

        def body(i, carry):
            off = base + i * _CHUNK
            pltpu.sync_copy(idx_hbm.at[pl.ds(off, _CHUNK)], idx_v)
            pltpu.async_copy(table_hbm.at[idx_v], rows_v, sem).wait()
            pltpu.sync_copy(rows_v, out_hbm.at[pl.ds(off, _CHUNK)])
            return carry

        lax.fori_loop(0, n_chunks, body, 0)

    return gather_kernel(idx_flat, table)


def kernel(x, table):
    batch, n_fields = x.shape
    _, embed_dim = table.shape
    n_rows = batch * n_fields
    flat = _sc_gather(x.reshape(-1), table, n_rows, embed_dim)
    return flat.reshape(batch, n_fields, embed_dim)


# speedup vs baseline: 1.0048x; 1.0048x over previous
import functools

import jax
import jax.numpy as jnp
from jax import lax
from jax.experimental import pallas as pl
from jax.experimental.pallas import tpu as pltpu
from jax.experimental.pallas import tpu_sc as plsc

_NUM_CORES = 2
_NUM_SUBCORES = 16
_NUM_WORKERS = _NUM_CORES * _NUM_SUBCORES
_CHUNK = 512


@functools.partial(jax.jit, static_argnames=("n_rows", "embed_dim"))
def _sc_gather(idx_flat, table, n_rows, embed_dim):
    rows_per_w = n_rows // _NUM_WORKERS
    n_chunks = rows_per_w // _CHUNK
    n_pairs = n_chunks // 2
    mesh = plsc.VectorSubcoreMesh(core_axis_name="c", subcore_axis_name="s")

    @functools.partial(
        pl.kernel,
        out_type=jax.ShapeDtypeStruct((n_rows, embed_dim), jnp.float32),
        mesh=mesh,
        scratch_types=[
            pltpu.VMEM((rows_per_w,), jnp.int32),
            pltpu.VMEM((_CHUNK, embed_dim), jnp.float32),
            pltpu.VMEM((_CHUNK, embed_dim), jnp.float32),
            pltpu.SemaphoreType.DMA,
            pltpu.SemaphoreType.DMA,
            pltpu.SemaphoreType.DMA,
            pltpu.SemaphoreType.DMA,
        ],
        compiler_params=pltpu.CompilerParams(use_tc_tiling_on_sc=False),
    )
    def gather_kernel(idx_hbm, table_hbm, out_hbm, idx_v, rows0, rows1,
                      sg0, sg1, ss0, ss1):
        wid = lax.axis_index("s") * _NUM_CORES + lax.axis_index("c")
        base = wid * rows_per_w
        pltpu.sync_copy(idx_hbm.at[pl.ds(base, rows_per_w)], idx_v)

        def gather_start(chunk, rows_v, sem):
            src = table_hbm.at[idx_v.at[pl.ds(chunk * _CHUNK, _CHUNK)]]
            pltpu.async_copy(src, rows_v, sem)

        def gather_wait(rows_v, sem):
            src = table_hbm.at[idx_v.at[pl.ds(0, _CHUNK)]]
            pltpu.make_async_copy(src, rows_v, sem).wait()

        def scatter_start(chunk, rows_v, sem):
            dst = out_hbm.at[pl.ds(base + chunk * _CHUNK, _CHUNK)]
            pltpu.async_copy(rows_v, dst, sem)

        def scatter_wait(rows_v, sem):
            dst = out_hbm.at[pl.ds(base, _CHUNK)]
            pltpu.make_async_copy(rows_v, dst, sem).wait()

        gather_start(0, rows0, sg0)
        gather_start(1, rows1, sg1)

        def body(p, carry):
            i0 = 2 * p
            gather_wait(rows0, sg0)
            scatter_start(i0, rows0, ss0)
            gather_wait(rows1, sg1)
            scatter_start(i0 + 1, rows1, ss1)
            @pl.when(p + 1 < n_pairs)
            def _():
                scatter_wait(rows0, ss0)
                gather_start(i0 + 2, rows0, sg0)
                scatter_wait(rows1, ss1)
                gather_start(i0 + 3, rows1, sg1)
            return carry

        lax.fori_loop(0, n_pairs, body, 0)
        scatter_wait(rows0, ss0)
        scatter_wait(rows1, ss1)

    return gather_kernel(idx_flat, table)


def kernel(x, table):
    batch, n_fields = x.shape
    _, embed_dim = table.shape
    n_rows = batch * n_fields
    flat = _sc_gather(x.reshape(-1), table, n_rows, embed_dim)
    return flat.reshape(batch, n_fields, embed_dim)
